# baseline (speedup 1.0000x reference)
import functools

import jax
import jax.numpy as jnp
from jax import lax
from jax.experimental import pallas as pl
from jax.experimental.pallas import tpu as pltpu
from jax.experimental.pallas import tpu_sc as plsc

_HIDDEN = 128
_PROJ = 64
_TAGS = 1000
_BATCH = 16384
_IDX_CHUNK = 128


@functools.cache
def _sc_gather_fn(n_rows, D, nc, ns):
    nw = nc * ns
    n_chunks_per_w = n_rows // (nw * _IDX_CHUNK)
    mesh = plsc.VectorSubcoreMesh(core_axis_name="c", subcore_axis_name="s")

    @functools.partial(
        pl.kernel,
        mesh=mesh,
        out_type=jax.ShapeDtypeStruct((n_rows // _IDX_CHUNK, _IDX_CHUNK, D),
                                      jnp.float32),
        scratch_types=[
            pltpu.VMEM((n_chunks_per_w, _IDX_CHUNK), jnp.int32),
            pltpu.VMEM((n_chunks_per_w, _IDX_CHUNK, D), jnp.float32),
            pltpu.SemaphoreType.DMA,
        ],
    )
    def gather_k(table_hbm, idx_hbm, out_hbm, idx_v, rows_v, sem):
        wid = lax.axis_index("s") * nc + lax.axis_index("c")
        base = wid * n_chunks_per_w
        pltpu.sync_copy(idx_hbm.at[pl.ds(base, n_chunks_per_w)], idx_v)
        copies = [
            pltpu.async_copy(table_hbm.at[idx_v.at[j]], rows_v.at[j], sem)
            for j in range(n_chunks_per_w)
        ]
        for c in copies:
            c.wait()
        pltpu.sync_copy(rows_v, out_hbm.at[pl.ds(base, n_chunks_per_w)])

    return gather_k


def _mm_compute(h_ref, e_ref, w1t_ref, w2t_ref, b_ref):
    h_bf = h_ref[...].astype(jnp.bfloat16)
    e_bf = e_ref[...].astype(jnp.bfloat16)
    acc = jnp.dot(h_bf, w1t_ref[...], preferred_element_type=jnp.float32)
    acc = acc + jnp.dot(e_bf, w2t_ref[...],
                        preferred_element_type=jnp.float32)
    return acc + b_ref[...]


def _mm_body_first(h_ref, e_ref, w1t_ref, w2t_ref, b_ref, o_ref):
    o_ref[...] = _mm_compute(h_ref, e_ref, w1t_ref, w2t_ref, b_ref)


def _mm_body_second(prev_ref, h_ref, e_ref, w1t_ref, w2t_ref, b_ref, o_ref):
    del prev_ref
    o_ref[...] = _mm_compute(h_ref, e_ref, w1t_ref, w2t_ref, b_ref)


def kernel(hidden, tag, is_train, tag_table, W, b):
    del is_train
    B, H = hidden.shape
    V, D = tag_table.shape
    T = W.shape[0]

    info = plsc.get_sparse_core_info()
    nc, ns = info.num_cores, info.num_subcores

    Dp = 128
    table_pad = jnp.pad(tag_table, ((0, 0), (0, Dp - D)))
    idx = tag.astype(jnp.int32).reshape(B // _IDX_CHUNK, _IDX_CHUNK)

    Wt = W.T.astype(jnp.bfloat16)
    w1t = Wt[:H]
    w2t = jnp.concatenate(
        [Wt[H:], jnp.zeros((Dp - D, T), jnp.bfloat16)], axis=0)
    b2 = b.reshape(1, T)

    NCHUNK = 2
    BT = 4096
    Bc = B // NCHUNK
    nblk = Bc // BT
    rows_per_chunk = Bc // _IDX_CHUNK
    gather = _sc_gather_fn(Bc, Dp, nc, ns)

    embs, hs = [], []
    for c in range(NCHUNK):
        idx_c = lax.slice_in_dim(idx, c * rows_per_chunk,
                                 (c + 1) * rows_per_chunk, axis=0)
        embs.append(gather(table_pad, idx_c).reshape(Bc, Dp))
        hs.append(lax.slice_in_dim(hidden, c * Bc, (c + 1) * Bc, axis=0))

    common_in_specs = [
        pl.BlockSpec((BT, H), lambda i: (i, 0)),
        pl.BlockSpec((BT, Dp), lambda i: (i, 0)),
        pl.BlockSpec((H, T), lambda i: (0, 0)),
        pl.BlockSpec((Dp, T), lambda i: (0, 0)),
        pl.BlockSpec((1, T), lambda i: (0, 0)),
    ]
    out_shape = jax.ShapeDtypeStruct((B, T), jnp.float32)

    out0 = pl.pallas_call(
        _mm_body_first,
        grid=(nblk,),
        in_specs=common_in_specs,
        out_specs=pl.BlockSpec((BT, T), lambda i: (i, 0)),
        out_shape=out_shape,
        compiler_params=pltpu.CompilerParams(
            dimension_semantics=("arbitrary",)),
    )(hs[0], embs[0], w1t, w2t, b2)

    out = pl.pallas_call(
        _mm_body_second,
        grid=(nblk,),
        in_specs=[pl.BlockSpec(memory_space=pl.ANY)] + common_in_specs,
        out_specs=pl.BlockSpec((BT, T), lambda i: (i + nblk, 0)),
        out_shape=out_shape,
        input_output_aliases={0: 0},
        compiler_params=pltpu.CompilerParams(
            dimension_semantics=("arbitrary",)),
    )(out0, hs[1], embs[1], w1t, w2t, b2)
    return out


# speedup vs baseline: 1.1185x; 1.1185x over previous
import functools

import jax
import jax.numpy as jnp
from jax import lax
from jax.experimental import pallas as pl
from jax.experimental.pallas import tpu as pltpu
from jax.experimental.pallas import tpu_sc as plsc

_IDX_CHUNK = 128


@functools.cache
def _sc_gather_fn(n_rows, D, nc, ns):
    nw = nc * ns
    n_chunks_per_w = n_rows // (nw * _IDX_CHUNK)
    mesh = plsc.VectorSubcoreMesh(core_axis_name="c", subcore_axis_name="s")

    @functools.partial(
        pl.kernel,
        mesh=mesh,
        out_type=jax.ShapeDtypeStruct((n_rows // _IDX_CHUNK, _IDX_CHUNK, D),
                                      jnp.float32),
        scratch_types=[
            pltpu.VMEM((n_chunks_per_w, _IDX_CHUNK), jnp.int32),
            pltpu.VMEM((n_chunks_per_w, _IDX_CHUNK, D), jnp.float32),
            pltpu.SemaphoreType.DMA,
        ],
    )
    def gather_k(table_hbm, idx_hbm, out_hbm, idx_v, rows_v, sem):
        wid = lax.axis_index("s") * nc + lax.axis_index("c")
        base = wid * n_chunks_per_w
        pltpu.sync_copy(idx_hbm.at[pl.ds(base, n_chunks_per_w)], idx_v)
        copies = [
            pltpu.async_copy(table_hbm.at[idx_v.at[j]], rows_v.at[j], sem)
            for j in range(n_chunks_per_w)
        ]
        for c in copies:
            c.wait()
        pltpu.sync_copy(rows_v, out_hbm.at[pl.ds(base, n_chunks_per_w)])

    return gather_k


def _mm_body(h_ref, e_ref, w1t_ref, w2t_ref, b_ref, o_ref):
    h_bf = h_ref[...].astype(jnp.bfloat16)
    e_bf = e_ref[...].astype(jnp.bfloat16)
    acc = jnp.dot(h_bf, w1t_ref[...], preferred_element_type=jnp.float32)
    acc = acc + jnp.dot(e_bf, w2t_ref[...],
                        preferred_element_type=jnp.float32)
    o_ref[...] = acc + b_ref[...]


def kernel(hidden, tag, is_train, tag_table, W, b):
    del is_train
    B, H = hidden.shape
    V, D = tag_table.shape
    T = W.shape[0]

    info = plsc.get_sparse_core_info()
    nc, ns = info.num_cores, info.num_subcores

    Dp = 128
    table_pad = jnp.pad(tag_table, ((0, 0), (0, Dp - D)))
    idx = tag.astype(jnp.int32).reshape(B // _IDX_CHUNK, _IDX_CHUNK)
    emb = _sc_gather_fn(B, Dp, nc, ns)(table_pad, idx).reshape(B, Dp)

    Wt = W.T.astype(jnp.bfloat16)
    w1t = Wt[:H]
    w2t = jnp.concatenate(
        [Wt[H:], jnp.zeros((Dp - D, T), jnp.bfloat16)], axis=0)
    b2 = b.reshape(1, T)

    BT = 2048
    return pl.pallas_call(
        _mm_body,
        grid=(B // BT,),
        in_specs=[
            pl.BlockSpec((BT, H), lambda i: (i, 0)),
            pl.BlockSpec((BT, Dp), lambda i: (i, 0)),
            pl.BlockSpec((H, T), lambda i: (0, 0)),
            pl.BlockSpec((Dp, T), lambda i: (0, 0)),
            pl.BlockSpec((1, T), lambda i: (0, 0)),
        ],
        out_specs=pl.BlockSpec((BT, T), lambda i: (i, 0)),
        out_shape=jax.ShapeDtypeStruct((B, T), jnp.float32),
        compiler_params=pltpu.CompilerParams(
            dimension_semantics=("arbitrary",)),
    )(hidden, emb, w1t, w2t, b2)
